# baseline (speedup 1.0000x reference)
import functools

import jax
import jax.numpy as jnp
from jax import lax
from jax.experimental import pallas as pl
from jax.experimental.pallas import tpu as pltpu
from jax.experimental.pallas import tpu_sc as plsc

N = 10000
E = 320000
D = 128

NC = 2
NS = 16
NW = NC * NS
CH = 128
NCH = 80
EPW = NCH * CH
EP = NW * EPW
PH = NCH // 2
NP = 10240
STRIPE = NP // NS
ZB = 128

_MESH = plsc.VectorSubcoreMesh(core_axis_name="c", subcore_axis_name="s")


@functools.partial(
    pl.kernel,
    out_type=jax.ShapeDtypeStruct((NC, NP, D), jnp.float32),
    mesh=_MESH,
    scratch_types=[
        pltpu.VMEM((NCH, CH), jnp.int32),
        pltpu.VMEM((CH, D), jnp.float32),
        pltpu.SemaphoreType.DMA,
        pltpu.VMEM_SHARED((NP, D), jnp.float32),
    ],
)
def _deg_kernel(col2d, ones128, zerosd, out, col_v, ones_v, dsem, deg_sh):
    c = lax.axis_index("c")
    s = lax.axis_index("s")
    w = c * NS + s
    for k in range(STRIPE // ZB):
        pltpu.sync_copy(zerosd, deg_sh.at[pl.ds(s * STRIPE + k * ZB, ZB)])
    pltpu.sync_copy(ones128, ones_v)
    pltpu.sync_copy(col2d.at[w], col_v)
    plsc.subcore_barrier()

    def fire(j, carry):
        pltpu.async_copy(ones_v, deg_sh.at[col_v.at[j]], dsem, add=True)
        return carry

    def drain(j, carry):
        pltpu.make_async_copy(ones_v, deg_sh.at[col_v.at[j]], dsem).wait()
        return carry

    lax.fori_loop(0, NCH, fire, 0)
    lax.fori_loop(0, NCH, drain, 0)
    plsc.subcore_barrier()
    pltpu.sync_copy(deg_sh.at[pl.ds(s * STRIPE, STRIPE)],
                    out.at[c, pl.ds(s * STRIPE, STRIPE)])


@functools.partial(
    pl.kernel,
    out_type=jax.ShapeDtypeStruct((NC, NP, D), jnp.float32),
    mesh=_MESH,
    scratch_types=[
        pltpu.VMEM((PH, CH), jnp.int32),
        pltpu.VMEM((PH, CH), jnp.int32),
        pltpu.VMEM((CH, D), jnp.float32),
        pltpu.VMEM((CH, D), jnp.float32),
        pltpu.SemaphoreType.DMA,
        pltpu.SemaphoreType.DMA,
        pltpu.SemaphoreType.DMA,
        pltpu.SemaphoreType.DMA,
        pltpu.VMEM_SHARED((NP, D), jnp.float32),
    ],
)
def _spmm_kernel(x2, row2d, col2d, zerosd, out,
                 row_v, col_v, buf0, buf1, sem0, sem1, ssem0, ssem1, agg_sh):
    c = lax.axis_index("c")
    s = lax.axis_index("s")
    w = c * NS + s
    for k in range(STRIPE // ZB):
        pltpu.sync_copy(zerosd, agg_sh.at[pl.ds(s * STRIPE + k * ZB, ZB)])
    plsc.subcore_barrier()

    for ph in range(NCH // PH):
        pltpu.sync_copy(row2d.at[w, pl.ds(ph * PH, PH)], row_v)
        pltpu.sync_copy(col2d.at[w, pl.ds(ph * PH, PH)], col_v)
        pltpu.async_copy(x2.at[col_v.at[0]], buf0, sem0)
        pltpu.async_copy(x2.at[col_v.at[1]], buf1, sem1)

        def body(t, carry):
            j0 = 2 * t
            pltpu.make_async_copy(x2.at[col_v.at[j0]], buf0, sem0).wait()
            pltpu.async_copy(buf0, agg_sh.at[row_v.at[j0]], ssem0, add=True)
            pltpu.make_async_copy(x2.at[col_v.at[j0 + 1]], buf1, sem1).wait()
            pltpu.async_copy(buf1, agg_sh.at[row_v.at[j0 + 1]], ssem1,
                             add=True)
            pltpu.make_async_copy(buf0, agg_sh.at[row_v.at[j0]], ssem0).wait()
            pltpu.async_copy(x2.at[col_v.at[j0 + 2]], buf0, sem0)
            pltpu.make_async_copy(buf1, agg_sh.at[row_v.at[j0 + 1]],
                                  ssem1).wait()
            pltpu.async_copy(x2.at[col_v.at[j0 + 3]], buf1, sem1)
            return carry

        lax.fori_loop(0, PH // 2 - 1, body, 0)
        j0 = PH - 2
        pltpu.make_async_copy(x2.at[col_v.at[j0]], buf0, sem0).wait()
        pltpu.async_copy(buf0, agg_sh.at[row_v.at[j0]], ssem0, add=True)
        pltpu.make_async_copy(x2.at[col_v.at[j0 + 1]], buf1, sem1).wait()
        pltpu.async_copy(buf1, agg_sh.at[row_v.at[j0 + 1]], ssem1, add=True)
        pltpu.make_async_copy(buf0, agg_sh.at[row_v.at[j0]], ssem0).wait()
        pltpu.make_async_copy(buf1, agg_sh.at[row_v.at[j0 + 1]], ssem1).wait()

    plsc.subcore_barrier()
    pltpu.sync_copy(agg_sh.at[pl.ds(s * STRIPE, STRIPE)],
                    out.at[c, pl.ds(s * STRIPE, STRIPE)])


def _prep_body(x_ref, deg_ref, p_ref, x2_ref, dis_ref, mu_ref):
    x = x_ref[...]
    mu = jnp.min(x)
    pp = jax.nn.sigmoid(p_ref[...][0, 0]) + 1.0
    deg = deg_ref[0, 0:N, 0:1] + deg_ref[1, 0:N, 0:1]
    dis = jnp.where(deg > 0.0, lax.rsqrt(deg), 0.0)
    x2_ref[0:N, :] = dis * jnp.power(x - mu + 1e-6, pp)
    x2_ref[N:NP, :] = jnp.zeros((NP - N, D), jnp.float32)
    dis_ref[...] = dis
    mu_ref[...] = jnp.reshape(mu, (1, 1))


_prep_call = pl.pallas_call(
    _prep_body,
    out_shape=(
        jax.ShapeDtypeStruct((NP, D), jnp.float32),
        jax.ShapeDtypeStruct((N, 1), jnp.float32),
        jax.ShapeDtypeStruct((1, 1), jnp.float32),
    ),
)


_RB = 2000


def _out_body(x_ref, agga_ref, aggb_ref, dis_ref, mu_ref, p_ref, eps_ref,
              o_ref):
    pp = jax.nn.sigmoid(p_ref[...][0, 0]) + 1.0
    mu = mu_ref[...][0, 0]
    eps = eps_ref[...][0, 0]
    agg = dis_ref[...] * (agga_ref[0] + aggb_ref[0])
    o_ref[...] = (jnp.power(agg + 1e-6, 1.0 / pp)
                  + (1.0 + eps) * x_ref[...] + mu)


_out_call = pl.pallas_call(
    _out_body,
    grid=(N // _RB,),
    in_specs=[
        pl.BlockSpec((_RB, D), lambda i: (i, 0)),
        pl.BlockSpec((1, _RB, D), lambda i: (0, i, 0)),
        pl.BlockSpec((1, _RB, D), lambda i: (1, i, 0)),
        pl.BlockSpec((_RB, 1), lambda i: (i, 0)),
        pl.BlockSpec((1, 1), lambda i: (0, 0)),
        pl.BlockSpec((1, 1), lambda i: (0, 0)),
        pl.BlockSpec((1, 1), lambda i: (0, 0)),
    ],
    out_specs=pl.BlockSpec((_RB, D), lambda i: (i, 0)),
    out_shape=jax.ShapeDtypeStruct((N, D), jnp.float32),
)


def kernel(x, edge_index, eps, p):
    pad = jnp.broadcast_to(N + (jnp.arange(EP - E, dtype=jnp.int32)
                                % (NP - N)), (2, EP - E))
    ei = jnp.concatenate([edge_index, pad], axis=1)
    row2d = ei[0].reshape(NW, NCH, CH)
    col2d = ei[1].reshape(NW, NCH, CH)
    ones128 = jnp.ones((CH, D), jnp.float32)
    zerosd = jnp.zeros((ZB, D), jnp.float32)
    p11 = p.reshape(1, 1)
    eps11 = eps.reshape(1, 1)

    deg2 = _deg_kernel(col2d, ones128, zerosd)
    x2, dis, mu = _prep_call(x, deg2, p11)
    agg2 = _spmm_kernel(x2, row2d, col2d, zerosd)
    return _out_call(x, agg2, agg2, dis, mu, p11, eps11)


# speedup vs baseline: 1.1463x; 1.1463x over previous
import functools

import jax
import jax.numpy as jnp
from jax import lax
from jax.experimental import pallas as pl
from jax.experimental.pallas import tpu as pltpu
from jax.experimental.pallas import tpu_sc as plsc

N = 10000
E = 320000
D = 128

NC = 2
NS = 16
NW = NC * NS
CH = 128
NCH = 80
EPW = NCH * CH
EP = NW * EPW
PH = NCH // 2
NP = 10240
STRIPE = NP // NS
ZB = 128

_MESH = plsc.VectorSubcoreMesh(core_axis_name="c", subcore_axis_name="s")


@functools.partial(
    pl.kernel,
    out_type=jax.ShapeDtypeStruct((NC, NP, D), jnp.float32),
    mesh=_MESH,
    scratch_types=[
        pltpu.VMEM((NCH, CH), jnp.int32),
        pltpu.VMEM((CH, D), jnp.float32),
        pltpu.SemaphoreType.DMA,
        pltpu.VMEM_SHARED((NP, D), jnp.float32),
    ],
)
def _deg_kernel(col2d, ones128, zerosd, out, col_v, ones_v, dsem, deg_sh):
    c = lax.axis_index("c")
    s = lax.axis_index("s")
    w = c * NS + s
    for k in range(STRIPE // ZB):
        pltpu.sync_copy(zerosd, deg_sh.at[pl.ds(s * STRIPE + k * ZB, ZB)])
    pltpu.sync_copy(ones128, ones_v)
    pltpu.sync_copy(col2d.at[w], col_v)
    plsc.subcore_barrier()

    def fire(j, carry):
        pltpu.async_copy(ones_v, deg_sh.at[col_v.at[j]], dsem, add=True)
        return carry

    def drain(j, carry):
        pltpu.make_async_copy(ones_v, deg_sh.at[col_v.at[j]], dsem).wait()
        return carry

    lax.fori_loop(0, NCH, fire, 0)
    lax.fori_loop(0, NCH, drain, 0)
    plsc.subcore_barrier()
    pltpu.sync_copy(deg_sh.at[pl.ds(s * STRIPE, STRIPE)],
                    out.at[c, pl.ds(s * STRIPE, STRIPE)])


@functools.partial(
    pl.kernel,
    out_type=jax.ShapeDtypeStruct((NC, NP, D), jnp.float32),
    mesh=_MESH,
    scratch_types=[
        pltpu.VMEM((PH, CH), jnp.int32),
        pltpu.VMEM((PH, CH), jnp.int32),
        pltpu.VMEM((CH, D), jnp.float32),
        pltpu.VMEM((CH, D), jnp.float32),
        pltpu.SemaphoreType.DMA,
        pltpu.SemaphoreType.DMA,
        pltpu.VMEM_SHARED((NP, D), jnp.float32),
    ],
)
def _spmm_kernel(x2, row2d, col2d, zerosd, out,
                 row_v, col_v, buf0, buf1, sem0, sem1, agg_sh):
    c = lax.axis_index("c")
    s = lax.axis_index("s")
    w = c * NS + s
    for k in range(STRIPE // ZB):
        pltpu.sync_copy(zerosd, agg_sh.at[pl.ds(s * STRIPE + k * ZB, ZB)])
    plsc.subcore_barrier()

    for ph in range(NCH // PH):
        pltpu.sync_copy(row2d.at[w, pl.ds(ph * PH, PH)], row_v)
        pltpu.sync_copy(col2d.at[w, pl.ds(ph * PH, PH)], col_v)
        pltpu.async_copy(x2.at[col_v.at[0]], buf0, sem0)

        def body(t, carry):
            j0 = 2 * t
            pltpu.async_copy(x2.at[col_v.at[j0 + 1]], buf1, sem1)
            pltpu.make_async_copy(x2.at[col_v.at[j0]], buf0, sem0).wait()
            pltpu.sync_copy(buf0, agg_sh.at[row_v.at[j0]], add=True)
            pltpu.async_copy(x2.at[col_v.at[j0 + 2]], buf0, sem0)
            pltpu.make_async_copy(x2.at[col_v.at[j0 + 1]], buf1, sem1).wait()
            pltpu.sync_copy(buf1, agg_sh.at[row_v.at[j0 + 1]], add=True)
            return carry

        lax.fori_loop(0, PH // 2 - 1, body, 0)
        pltpu.async_copy(x2.at[col_v.at[PH - 1]], buf1, sem1)
        pltpu.make_async_copy(x2.at[col_v.at[PH - 2]], buf0, sem0).wait()
        pltpu.sync_copy(buf0, agg_sh.at[row_v.at[PH - 2]], add=True)
        pltpu.make_async_copy(x2.at[col_v.at[PH - 1]], buf1, sem1).wait()
        pltpu.sync_copy(buf1, agg_sh.at[row_v.at[PH - 1]], add=True)

    plsc.subcore_barrier()
    pltpu.sync_copy(agg_sh.at[pl.ds(s * STRIPE, STRIPE)],
                    out.at[c, pl.ds(s * STRIPE, STRIPE)])


def _prep_body(x_ref, deg_ref, p_ref, x2_ref, dis_ref, mu_ref):
    x = x_ref[...]
    mu = jnp.min(x)
    pp = jax.nn.sigmoid(p_ref[...][0, 0]) + 1.0
    deg = deg_ref[0, 0:N, 0:1] + deg_ref[1, 0:N, 0:1]
    dis = jnp.where(deg > 0.0, lax.rsqrt(deg), 0.0)
    x2_ref[0:N, :] = dis * jnp.power(x - mu + 1e-6, pp)
    x2_ref[N:NP, :] = jnp.zeros((NP - N, D), jnp.float32)
    dis_ref[...] = dis
    mu_ref[...] = jnp.reshape(mu, (1, 1))


_prep_call = pl.pallas_call(
    _prep_body,
    out_shape=(
        jax.ShapeDtypeStruct((NP, D), jnp.float32),
        jax.ShapeDtypeStruct((N, 1), jnp.float32),
        jax.ShapeDtypeStruct((1, 1), jnp.float32),
    ),
)


_RB = 2000


def _out_body(x_ref, agga_ref, aggb_ref, dis_ref, mu_ref, p_ref, eps_ref,
              o_ref):
    pp = jax.nn.sigmoid(p_ref[...][0, 0]) + 1.0
    mu = mu_ref[...][0, 0]
    eps = eps_ref[...][0, 0]
    agg = dis_ref[...] * (agga_ref[0] + aggb_ref[0])
    o_ref[...] = (jnp.power(agg + 1e-6, 1.0 / pp)
                  + (1.0 + eps) * x_ref[...] + mu)


_out_call = pl.pallas_call(
    _out_body,
    grid=(N // _RB,),
    in_specs=[
        pl.BlockSpec((_RB, D), lambda i: (i, 0)),
        pl.BlockSpec((1, _RB, D), lambda i: (0, i, 0)),
        pl.BlockSpec((1, _RB, D), lambda i: (1, i, 0)),
        pl.BlockSpec((_RB, 1), lambda i: (i, 0)),
        pl.BlockSpec((1, 1), lambda i: (0, 0)),
        pl.BlockSpec((1, 1), lambda i: (0, 0)),
        pl.BlockSpec((1, 1), lambda i: (0, 0)),
    ],
    out_specs=pl.BlockSpec((_RB, D), lambda i: (i, 0)),
    out_shape=jax.ShapeDtypeStruct((N, D), jnp.float32),
)


def kernel(x, edge_index, eps, p):
    pad = jnp.broadcast_to(N + (jnp.arange(EP - E, dtype=jnp.int32)
                                % (NP - N)), (2, EP - E))
    ei = jnp.concatenate([edge_index, pad], axis=1)
    row2d = ei[0].reshape(NW, NCH, CH)
    col2d = ei[1].reshape(NW, NCH, CH)
    ones128 = jnp.ones((CH, D), jnp.float32)
    zerosd = jnp.zeros((ZB, D), jnp.float32)
    p11 = p.reshape(1, 1)
    eps11 = eps.reshape(1, 1)

    deg2 = _deg_kernel(col2d, ones128, zerosd)
    x2, dis, mu = _prep_call(x, deg2, p11)
    agg2 = _spmm_kernel(x2, row2d, col2d, zerosd)
    return _out_call(x, agg2, agg2, dis, mu, p11, eps11)
